# baseline (speedup 1.0000x reference)
import functools

import jax
import jax.numpy as jnp
from jax import lax
from jax.experimental import pallas as pl
from jax.experimental.pallas import tpu as pltpu
from jax.experimental.pallas import tpu_sc as plsc

BATCH = 16384
DIM = 16
V = 1000000
NW = 32
SHARD = 32768
NSHARD = 31
LAST_SZ = V - 30 * SHARD
CK = 4096
QCAP = 320
HCAP = 4 * QCAP
SCAN_T = BATCH // 16

_CP = pltpu.CompilerParams(needs_layout_passes=False)
_MESH = dict(core_axis_name="c", subcore_axis_name="s")


def _k1_body(sidx_hbm, aidx_hbm, pt_hbm, qt_hbm,
             pv0, pv1, pv2, pv3, pv4, pv5, pv6, pv7,
             pv8, pv9, pv10, pv11, pv12, pv13, pv14, pv15,
             qv0, qv1, qv2, qv3, qv4, qv5, qv6, qv7,
             qv8, qv9, qv10, qv11, qv12, qv13, qv14, qv15,
             pinv_hbm, qinv_hbm,
             sidx_v, aidx_v, hpi_v, hpj_v, hqi_v, hqj_v,
             chunk_v, tail_v, vals_v, kidx_v, sem):
    pvs = [pv0, pv1, pv2, pv3, pv4, pv5, pv6, pv7,
           pv8, pv9, pv10, pv11, pv12, pv13, pv14, pv15]
    qvs = [qv0, qv1, qv2, qv3, qv4, qv5, qv6, qv7,
           qv8, qv9, qv10, qv11, qv12, qv13, qv14, qv15]
    w = lax.axis_index("s") * 2 + lax.axis_index("c")
    shard_lo = w * SHARD
    shard_sz = jnp.where(w == 30, LAST_SZ, jnp.where(w >= NSHARD, 0, SHARD))

    pltpu.sync_copy(sidx_hbm, sidx_v)
    pltpu.sync_copy(aidx_hbm, aidx_v)

    zeros16 = jnp.zeros((16,), jnp.int32)

    def jinit(t, carry):
        dump16 = BATCH + w * HCAP + t * 16 + lax.iota(jnp.int32, 16)
        hpj_v[pl.ds(t * 16, 16)] = dump16
        hqj_v[pl.ds(t * 16, 16)] = dump16
        hpi_v[pl.ds(t * 16, 16)] = zeros16
        hqi_v[pl.ds(t * 16, 16)] = zeros16
        return carry
    lax.fori_loop(0, HCAP // 16, jinit, 0)

    QT = SCAN_T // 4

    def scan_body(t, carry):
        offs = list(carry)
        for qi in range(4):
            tt = t + qi * QT
            j16 = tt * 16 + lax.iota(jnp.int32, 16)
            si = sidx_v[pl.ds(tt * 16, 16)]
            ai = aidx_v[pl.ds(tt * 16, 16)]
            rp = si - shard_lo
            rq = ai - shard_lo
            mp = (rp >= 0) & (rp < shard_sz)
            mq = (rq >= 0) & (rq < shard_sz)
            plsc.store_compressed(hpi_v.at[pl.ds(offs[qi], 16)], rp,
                                  mask=mp)
            plsc.store_compressed(hpj_v.at[pl.ds(offs[qi], 16)], j16,
                                  mask=mp)
            plsc.store_compressed(hqi_v.at[pl.ds(offs[4 + qi], 16)], rq,
                                  mask=mq)
            plsc.store_compressed(hqj_v.at[pl.ds(offs[4 + qi], 16)], j16,
                                  mask=mq)
            np_ = lax.reduce_max(
                plsc.all_reduce_population_count(mp), axes=(0,))
            nq_ = lax.reduce_max(
                plsc.all_reduce_population_count(mq), axes=(0,))
            offs[qi] = offs[qi] + np_
            offs[4 + qi] = offs[4 + qi] + nq_
        return tuple(offs)

    inits = tuple(qi * QCAP for qi in range(4)) * 2
    offs = lax.fori_loop(0, QT, scan_body, inits)
    offps = [jnp.minimum(offs[qi], qi * QCAP + QCAP - 16)
             for qi in range(4)]
    offqs = [jnp.minimum(offs[4 + qi], qi * QCAP + QCAP - 16)
             for qi in range(4)]

    zeros = jnp.zeros((16,), jnp.int32)
    for qi in range(4):
        hpi_v[pl.ds(offps[qi], 16)] = zeros
        hqi_v[pl.ds(offqs[qi], 16)] = zeros
    nvps = [(offps[qi] - qi * QCAP + 15) // 16 + 1 for qi in range(4)]
    nvqs = [(offqs[qi] - qi * QCAP + 15) // 16 + 1 for qi in range(4)]

    def kfill(t, carry):
        kidx_v[pl.ds(t * 16, 16)] = w * HCAP + t * 16 + lax.iota(jnp.int32, 16)
        return carry
    lax.fori_loop(0, HCAP // 16, kfill, 0)
    cpi = pltpu.async_copy(kidx_v, pinv_hbm.at[hpj_v], sem)
    cpi.wait()
    cqi = pltpu.async_copy(kidx_v, qinv_hbm.at[hqj_v], sem)
    cqi.wait()

    dts = [jnp.full((16,), d >> 3, jnp.int32) for d in range(DIM)]
    dss = [jnp.full((16,), d & 7, jnp.int32) for d in range(DIM)]
    dcols16 = [jnp.full((16,), d, jnp.int32) for d in range(DIM)]

    def do_table(tbl_hbm, hi_v, hj_v, nvs, outs):
        nchunks = jnp.where(w == 30, 4, jnp.where(w >= NSHARD, 0, 8))

        def extract(c):
            def hit_body(t, carry):
                ri = hi_v[pl.ds(t * 16, 16)]
                inck = (ri // CK) == c
                lane = ri - c * CK
                lane = jnp.where(inck, lane, 0)
                slot = t * 16 + lax.iota(jnp.int32, 16)
                for d in range(DIM):
                    vd = plsc.load_gather(
                        chunk_v, [dts[d], dss[d], lane], mask=inck)
                    plsc.store_scatter(vals_v, [dcols16[d], slot], vd,
                                       mask=inck)
                return carry
            for qi in range(4):
                s0 = qi * (QCAP // 16)
                lax.fori_loop(s0, s0 + nvs[qi], hit_body, 0)

        def chunk_body(c, carry):
            start = pl.multiple_of(shard_lo + c * CK, 128)
            c0 = pltpu.async_copy(
                tbl_hbm.at[0].at[:, pl.ds(start, CK)], chunk_v.at[0], sem)
            c1 = pltpu.async_copy(
                tbl_hbm.at[1].at[:, pl.ds(start, CK)], chunk_v.at[1], sem)
            c0.wait()
            c1.wait()
            extract(c)
            return carry
        lax.fori_loop(0, nchunks, chunk_body, 0)

        @pl.when(w == 30)
        def _():
            for dt in range(2):
                pltpu.sync_copy(tbl_hbm.at[dt].at[:, pl.ds(999424, 512)],
                                chunk_v.at[dt].at[:, pl.ds(0, 512)])
                pltpu.sync_copy(tbl_hbm.at[dt].at[:, pl.ds(999936, 64)],
                                tail_v.at[dt])

            def hit_body(t, carry):
                ri = hi_v[pl.ds(t * 16, 16)]
                lane = ri - 4 * CK
                m1 = (lane >= 0) & (lane < 512)
                m2 = (lane >= 512) & (lane < 576)
                l1 = jnp.where(m1, lane, 0)
                l2 = jnp.where(m2, lane - 512, 0)
                slot = t * 16 + lax.iota(jnp.int32, 16)
                for d in range(DIM):
                    v1 = plsc.load_gather(
                        chunk_v, [dts[d], dss[d], l1], mask=m1)
                    plsc.store_scatter(vals_v, [dcols16[d], slot], v1,
                                       mask=m1)
                    v2 = plsc.load_gather(
                        tail_v, [dts[d], dss[d], l2], mask=m2)
                    plsc.store_scatter(vals_v, [dcols16[d], slot], v2,
                                       mask=m2)
                return carry
            for qi in range(4):
                s0 = qi * (QCAP // 16)
                lax.fori_loop(s0, s0 + nvs[qi], hit_body, 0)

        for d in range(DIM):
            pltpu.sync_copy(vals_v.at[d].at[pl.ds(0, HCAP)],
                            outs[d].at[pl.ds(w * HCAP, HCAP)])

    do_table(pt_hbm, hpi_v, hpj_v, nvps, pvs)
    do_table(qt_hbm, hqi_v, hqj_v, nvqs, qvs)


def _make_k1():
    out_type = ([jax.ShapeDtypeStruct((NW * HCAP,), jnp.float32)] * 32
                + [jax.ShapeDtypeStruct((BATCH + NW * HCAP,), jnp.int32)] * 2)
    scratch = [
        pltpu.VMEM((BATCH,), jnp.int32),
        pltpu.VMEM((BATCH,), jnp.int32),
        pltpu.VMEM((HCAP + 32,), jnp.int32),
        pltpu.VMEM((HCAP,), jnp.int32),
        pltpu.VMEM((HCAP + 32,), jnp.int32),
        pltpu.VMEM((HCAP,), jnp.int32),
        pltpu.VMEM((2, 8, CK), jnp.float32),
        pltpu.VMEM((2, 8, 64), jnp.float32),
        pltpu.VMEM((DIM, HCAP + 32), jnp.float32),
        pltpu.VMEM((HCAP,), jnp.int32),
        pltpu.SemaphoreType.DMA,
    ]
    return functools.partial(
        pl.kernel, out_type=out_type,
        mesh=plsc.VectorSubcoreMesh(**_MESH),
        compiler_params=_CP, scratch_types=scratch)(_k1_body)


def _k2_impl(pinv_hbm, qinv_hbm, sidx_hbm, aidx_hbm, pb_hbm, qb_hbm,
             *refs):
    pvs = refs[0:16]
    qvs = refs[16:32]
    out_hbm = refs[32]
    (pinv_v, qinv_v, sidx_v, aidx_v, pst_v, qst_v, pb_v, qb_v,
     out_v, sem) = refs[33:]
    w = lax.axis_index("s") * 2 + lax.axis_index("c")
    base = w * (BATCH // NW)
    n = BATCH // NW

    pltpu.sync_copy(pinv_hbm.at[pl.ds(base, n)], pinv_v)
    pltpu.sync_copy(qinv_hbm.at[pl.ds(base, n)], qinv_v)
    pltpu.sync_copy(sidx_hbm.at[pl.ds(base, n)], sidx_v)
    pltpu.sync_copy(aidx_hbm.at[pl.ds(base, n)], aidx_v)

    cps = []
    for d in range(DIM):
        cps.append(pltpu.async_copy(
            pvs[d].at[pinv_v], pst_v.at[pl.ds(d * n, n)], sem))
        cps.append(pltpu.async_copy(
            qvs[d].at[qinv_v], qst_v.at[pl.ds(d * n, n)], sem))
    cps.append(pltpu.async_copy(pb_hbm.at[sidx_v], pb_v, sem))
    cps.append(pltpu.async_copy(qb_hbm.at[aidx_v], qb_v, sem))
    for cp in cps:
        cp.wait()

    def tile(t, carry):
        acc = pb_v[pl.ds(t * 16, 16)] + qb_v[pl.ds(t * 16, 16)]
        for d in range(DIM):
            acc = acc + (pst_v[pl.ds(d * n + t * 16, 16)]
                         * qst_v[pl.ds(d * n + t * 16, 16)])
        out_v[pl.ds(t * 16, 16)] = acc
        return carry
    lax.fori_loop(0, n // 16, tile, 0)
    pltpu.sync_copy(out_v, out_hbm.at[pl.ds(base, n)])


def _make_k2():
    n = BATCH // NW
    scratch = [
        pltpu.VMEM((n,), jnp.int32),
        pltpu.VMEM((n,), jnp.int32),
        pltpu.VMEM((n,), jnp.int32),
        pltpu.VMEM((n,), jnp.int32),
        pltpu.VMEM((DIM * n,), jnp.float32),
        pltpu.VMEM((DIM * n,), jnp.float32),
        pltpu.VMEM((n,), jnp.float32),
        pltpu.VMEM((n,), jnp.float32),
        pltpu.VMEM((n,), jnp.float32),
        pltpu.SemaphoreType.DMA,
    ]
    return functools.partial(
        pl.kernel, out_type=jax.ShapeDtypeStruct((BATCH,), jnp.float32),
        mesh=plsc.VectorSubcoreMesh(**_MESH),
        compiler_params=_CP, scratch_types=scratch)(_k2_impl)


@jax.jit
def _mf(sidx, aidx, P, Q, Pb, Qb):
    pt = jnp.swapaxes(P, 0, 1).reshape(2, 8, V)
    qt = jnp.swapaxes(Q, 0, 1).reshape(2, 8, V)
    k1 = _make_k1()
    outs = k1(sidx, aidx, pt, qt)
    pvs, qvs = outs[0:16], outs[16:32]
    pinv, qinv = outs[32], outs[33]
    k2 = _make_k2()
    return k2(pinv[:BATCH], qinv[:BATCH], sidx, aidx,
              Pb.reshape(-1), Qb.reshape(-1), *pvs, *qvs)


def kernel(skill_sequence, attempt_sequence, P, Q, P_bias, Q_bias):
    out = _mf(skill_sequence.astype(jnp.int32),
              attempt_sequence.astype(jnp.int32), P, Q, P_bias, Q_bias)
    return out.reshape(BATCH, 1)


# speedup vs baseline: 1.1413x; 1.1413x over previous
import functools

import jax
import jax.numpy as jnp
from jax import lax
from jax.experimental import pallas as pl
from jax.experimental.pallas import tpu as pltpu
from jax.experimental.pallas import tpu_sc as plsc

BATCH = 16384
DIM = 16
V = 1000000
NW = 32
SHARD = 32768
NSHARD = 31
LAST_SZ = V - 30 * SHARD
CK = 4096
HCAP = 1024
SCAN_T = BATCH // 16

_CP = pltpu.CompilerParams(needs_layout_passes=False)
_MESH = dict(core_axis_name="c", subcore_axis_name="s")


def _k1_body(sidx_hbm, aidx_hbm, pt_hbm, qt_hbm,
             pv0, pv1, pv2, pv3, pv4, pv5, pv6, pv7,
             pv8, pv9, pv10, pv11, pv12, pv13, pv14, pv15,
             qv0, qv1, qv2, qv3, qv4, qv5, qv6, qv7,
             qv8, qv9, qv10, qv11, qv12, qv13, qv14, qv15,
             pinv_hbm, qinv_hbm,
             sidx_v, aidx_v, hpi_v, hpj_v, hqi_v, hqj_v,
             chunk_v, tail_v, vals_v, kidx_v, sem):
    pvs = [pv0, pv1, pv2, pv3, pv4, pv5, pv6, pv7,
           pv8, pv9, pv10, pv11, pv12, pv13, pv14, pv15]
    qvs = [qv0, qv1, qv2, qv3, qv4, qv5, qv6, qv7,
           qv8, qv9, qv10, qv11, qv12, qv13, qv14, qv15]
    w = lax.axis_index("s") * 2 + lax.axis_index("c")
    shard_lo = w * SHARD
    shard_sz = jnp.where(w == 30, LAST_SZ, jnp.where(w >= NSHARD, 0, SHARD))

    pltpu.sync_copy(sidx_hbm, sidx_v)
    pltpu.sync_copy(aidx_hbm, aidx_v)

    def jinit(t, carry):
        dump16 = BATCH + w * HCAP + t * 16 + lax.iota(jnp.int32, 16)
        hpj_v[pl.ds(t * 16, 16)] = dump16
        hqj_v[pl.ds(t * 16, 16)] = dump16
        return carry
    lax.fori_loop(0, HCAP // 16, jinit, 0)

    def scan_body(t, carry):
        offp, offq = carry
        j16 = t * 16 + lax.iota(jnp.int32, 16)
        si = sidx_v[pl.ds(t * 16, 16)]
        ai = aidx_v[pl.ds(t * 16, 16)]
        rp = si - shard_lo
        rq = ai - shard_lo
        mp = (rp >= 0) & (rp < shard_sz)
        mq = (rq >= 0) & (rq < shard_sz)
        plsc.store_compressed(hpi_v.at[pl.ds(offp, 16)], rp, mask=mp)
        plsc.store_compressed(hpj_v.at[pl.ds(offp, 16)], j16, mask=mp)
        plsc.store_compressed(hqi_v.at[pl.ds(offq, 16)], rq, mask=mq)
        plsc.store_compressed(hqj_v.at[pl.ds(offq, 16)], j16, mask=mq)
        np_ = lax.reduce_max(plsc.all_reduce_population_count(mp), axes=(0,))
        nq_ = lax.reduce_max(plsc.all_reduce_population_count(mq), axes=(0,))
        return offp + np_, offq + nq_

    offp, offq = lax.fori_loop(0, SCAN_T, scan_body, (0, 0))
    offp = jnp.minimum(offp, HCAP - 16)
    offq = jnp.minimum(offq, HCAP - 16)

    zeros = jnp.zeros((16,), jnp.int32)
    hpi_v[pl.ds(offp, 16)] = zeros
    hqi_v[pl.ds(offq, 16)] = zeros
    nvp = (offp + 15) // 16 + 1
    nvq = (offq + 15) // 16 + 1

    def kfill(t, carry):
        kidx_v[pl.ds(t * 16, 16)] = w * HCAP + t * 16 + lax.iota(jnp.int32, 16)
        return carry
    lax.fori_loop(0, HCAP // 16, kfill, 0)
    cpi = pltpu.async_copy(kidx_v, pinv_hbm.at[hpj_v], sem)
    cpi.wait()
    cqi = pltpu.async_copy(kidx_v, qinv_hbm.at[hqj_v], sem)
    cqi.wait()

    dts = [jnp.full((16,), d >> 3, jnp.int32) for d in range(DIM)]
    dss = [jnp.full((16,), d & 7, jnp.int32) for d in range(DIM)]
    dcols16 = [jnp.full((16,), d, jnp.int32) for d in range(DIM)]

    def do_table(tbl_hbm, hi_v, hj_v, nv, outs):
        nchunks = jnp.where(w == 30, 4, jnp.where(w >= NSHARD, 0, 8))

        def extract(c):
            def hit_body(t, carry):
                ri = hi_v[pl.ds(t * 16, 16)]
                inck = (ri // CK) == c
                lane = ri - c * CK
                lane = jnp.where(inck, lane, 0)
                slot = t * 16 + lax.iota(jnp.int32, 16)
                for d in range(DIM):
                    vd = plsc.load_gather(
                        chunk_v, [dts[d], dss[d], lane], mask=inck)
                    plsc.store_scatter(vals_v, [dcols16[d], slot], vd,
                                       mask=inck)
                return carry
            lax.fori_loop(0, nv, hit_body, 0)

        def chunk_body(c, carry):
            start = pl.multiple_of(shard_lo + c * CK, 128)
            c0 = pltpu.async_copy(
                tbl_hbm.at[0].at[:, pl.ds(start, CK)], chunk_v.at[0], sem)
            c1 = pltpu.async_copy(
                tbl_hbm.at[1].at[:, pl.ds(start, CK)], chunk_v.at[1], sem)
            c0.wait()
            c1.wait()
            extract(c)
            return carry
        lax.fori_loop(0, nchunks, chunk_body, 0)

        @pl.when(w == 30)
        def _():
            for dt in range(2):
                pltpu.sync_copy(tbl_hbm.at[dt].at[:, pl.ds(999424, 512)],
                                chunk_v.at[dt].at[:, pl.ds(0, 512)])
                pltpu.sync_copy(tbl_hbm.at[dt].at[:, pl.ds(999936, 64)],
                                tail_v.at[dt])

            def hit_body(t, carry):
                ri = hi_v[pl.ds(t * 16, 16)]
                lane = ri - 4 * CK
                m1 = (lane >= 0) & (lane < 512)
                m2 = (lane >= 512) & (lane < 576)
                l1 = jnp.where(m1, lane, 0)
                l2 = jnp.where(m2, lane - 512, 0)
                slot = t * 16 + lax.iota(jnp.int32, 16)
                for d in range(DIM):
                    v1 = plsc.load_gather(
                        chunk_v, [dts[d], dss[d], l1], mask=m1)
                    plsc.store_scatter(vals_v, [dcols16[d], slot], v1,
                                       mask=m1)
                    v2 = plsc.load_gather(
                        tail_v, [dts[d], dss[d], l2], mask=m2)
                    plsc.store_scatter(vals_v, [dcols16[d], slot], v2,
                                       mask=m2)
                return carry
            lax.fori_loop(0, nv, hit_body, 0)

        for d in range(DIM):
            pltpu.sync_copy(vals_v.at[d].at[pl.ds(0, HCAP)],
                            outs[d].at[pl.ds(w * HCAP, HCAP)])

    do_table(pt_hbm, hpi_v, hpj_v, nvp, pvs)
    do_table(qt_hbm, hqi_v, hqj_v, nvq, qvs)


def _make_k1():
    out_type = ([jax.ShapeDtypeStruct((NW * HCAP,), jnp.float32)] * 32
                + [jax.ShapeDtypeStruct((BATCH + NW * HCAP,), jnp.int32)] * 2)
    scratch = [
        pltpu.VMEM((BATCH,), jnp.int32),
        pltpu.VMEM((BATCH,), jnp.int32),
        pltpu.VMEM((HCAP + 32,), jnp.int32),
        pltpu.VMEM((HCAP,), jnp.int32),
        pltpu.VMEM((HCAP + 32,), jnp.int32),
        pltpu.VMEM((HCAP,), jnp.int32),
        pltpu.VMEM((2, 8, CK), jnp.float32),
        pltpu.VMEM((2, 8, 64), jnp.float32),
        pltpu.VMEM((DIM, HCAP + 32), jnp.float32),
        pltpu.VMEM((HCAP,), jnp.int32),
        pltpu.SemaphoreType.DMA,
    ]
    return functools.partial(
        pl.kernel, out_type=out_type,
        mesh=plsc.VectorSubcoreMesh(**_MESH),
        compiler_params=_CP, scratch_types=scratch)(_k1_body)


def _k2_impl(pinv_hbm, qinv_hbm, sidx_hbm, aidx_hbm, pb_hbm, qb_hbm,
             *refs):
    pvs = refs[0:16]
    qvs = refs[16:32]
    out_hbm = refs[32]
    (pinv_v, qinv_v, sidx_v, aidx_v, pst_v, qst_v, pb_v, qb_v,
     out_v, sem) = refs[33:]
    w = lax.axis_index("s") * 2 + lax.axis_index("c")
    base = w * (BATCH // NW)
    n = BATCH // NW

    pltpu.sync_copy(pinv_hbm.at[pl.ds(base, n)], pinv_v)
    pltpu.sync_copy(qinv_hbm.at[pl.ds(base, n)], qinv_v)
    pltpu.sync_copy(sidx_hbm.at[pl.ds(base, n)], sidx_v)
    pltpu.sync_copy(aidx_hbm.at[pl.ds(base, n)], aidx_v)

    cps = []
    for d in range(DIM):
        cps.append(pltpu.async_copy(
            pvs[d].at[pinv_v], pst_v.at[pl.ds(d * n, n)], sem))
        cps.append(pltpu.async_copy(
            qvs[d].at[qinv_v], qst_v.at[pl.ds(d * n, n)], sem))
    cps.append(pltpu.async_copy(pb_hbm.at[sidx_v], pb_v, sem))
    cps.append(pltpu.async_copy(qb_hbm.at[aidx_v], qb_v, sem))
    for cp in cps:
        cp.wait()

    def tile(t, carry):
        acc = pb_v[pl.ds(t * 16, 16)] + qb_v[pl.ds(t * 16, 16)]
        for d in range(DIM):
            acc = acc + (pst_v[pl.ds(d * n + t * 16, 16)]
                         * qst_v[pl.ds(d * n + t * 16, 16)])
        out_v[pl.ds(t * 16, 16)] = acc
        return carry
    lax.fori_loop(0, n // 16, tile, 0)
    pltpu.sync_copy(out_v, out_hbm.at[pl.ds(base, n)])


def _make_k2():
    n = BATCH // NW
    scratch = [
        pltpu.VMEM((n,), jnp.int32),
        pltpu.VMEM((n,), jnp.int32),
        pltpu.VMEM((n,), jnp.int32),
        pltpu.VMEM((n,), jnp.int32),
        pltpu.VMEM((DIM * n,), jnp.float32),
        pltpu.VMEM((DIM * n,), jnp.float32),
        pltpu.VMEM((n,), jnp.float32),
        pltpu.VMEM((n,), jnp.float32),
        pltpu.VMEM((n,), jnp.float32),
        pltpu.SemaphoreType.DMA,
    ]
    return functools.partial(
        pl.kernel, out_type=jax.ShapeDtypeStruct((BATCH,), jnp.float32),
        mesh=plsc.VectorSubcoreMesh(**_MESH),
        compiler_params=_CP, scratch_types=scratch)(_k2_impl)


@jax.jit
def _mf(sidx, aidx, P, Q, Pb, Qb):
    pt = jnp.swapaxes(P, 0, 1).reshape(2, 8, V)
    qt = jnp.swapaxes(Q, 0, 1).reshape(2, 8, V)
    k1 = _make_k1()
    outs = k1(sidx, aidx, pt, qt)
    pvs, qvs = outs[0:16], outs[16:32]
    pinv, qinv = outs[32], outs[33]
    k2 = _make_k2()
    return k2(pinv[:BATCH], qinv[:BATCH], sidx, aidx,
              Pb.reshape(-1), Qb.reshape(-1), *pvs, *qvs)


def kernel(skill_sequence, attempt_sequence, P, Q, P_bias, Q_bias):
    out = _mf(skill_sequence.astype(jnp.int32),
              attempt_sequence.astype(jnp.int32), P, Q, P_bias, Q_bias)
    return out.reshape(BATCH, 1)
